# baseline (speedup 1.0000x reference)
import functools

import jax
import jax.numpy as jnp
from jax import lax
from jax.experimental import pallas as pl
from jax.experimental.pallas import tpu as pltpu
from jax.experimental.pallas import tpu_sc as plsc

D_MODEL = 64
NC, NS = 2, 16
NW = NC * NS
NBUF = 4
LANES = 16


def kernel(inputs, token_table, pos_table):
    nseq, slen = inputs.shape
    bpw = nseq // NW
    ids_per_w = bpw * slen
    groups = slen // NBUF
    kb = bpw // LANES
    mesh = plsc.VectorSubcoreMesh(core_axis_name="c", subcore_axis_name="s")

    @functools.partial(
        pl.kernel,
        out_type=jax.ShapeDtypeStruct((slen, D_MODEL, nseq), jnp.float32),
        mesh=mesh,
        scratch_types=(
            [pltpu.VMEM((ids_per_w,), jnp.int32),
             pltpu.VMEM((slen, bpw), jnp.int32),
             pltpu.VMEM((slen * D_MODEL,), jnp.float32)]
            + [pltpu.VMEM((bpw, D_MODEL), jnp.float32)] * NBUF
            + [pltpu.VMEM((D_MODEL, bpw), jnp.float32)] * NBUF
            + [pltpu.SemaphoreType.DMA] * (2 * NBUF)
        ),
        compiler_params=pltpu.CompilerParams(use_tc_tiling_on_sc=False,
                                             needs_layout_passes=False),
    )
    def k(ids_hbm, tok_tab_hbm, pos_tab_hbm, out_hbm, ti, tix, pos_v, *rest):
        gbufs = rest[:NBUF]
        tbufs = rest[NBUF:2 * NBUF]
        sem_g = rest[2 * NBUF:3 * NBUF]
        sem_s = rest[3 * NBUF:]
        wid = lax.axis_index("s") * NC + lax.axis_index("c")
        b0 = wid * bpw

        pltpu.sync_copy(ids_hbm.at[pl.ds(b0 * slen, ids_per_w)], ti)
        pltpu.sync_copy(pos_tab_hbm.at[pl.ds(0, slen * D_MODEL)], pos_v)

        iota = lax.iota(jnp.int32, LANES)
        lanes_k = [iota + LANES * kk for kk in range(kb)]
        seqstr_k = [v * slen for v in lanes_k]
        flatg_k = [v * D_MODEL for v in lanes_k]

        @plsc.parallel_loop(0, slen, unroll=4)
        def idtr(l):
            for kk in range(kb):
                col = plsc.load_gather(ti, [seqstr_k[kk] + l])
                tix[l, pl.ds(kk * LANES, LANES)] = col

        def group(g, carry):
            descs = []
            for s in range(NBUF):
                l = g * NBUF + s

                @pl.when(g > 0)
                def _wait_prev_store(s=s):
                    pltpu.make_async_copy(
                        tbufs[s], out_hbm.at[0, :, pl.ds(0, bpw)],
                        sem_s[s]).wait()

                descs.append(
                    pltpu.async_copy(tok_tab_hbm.at[tix.at[l]], gbufs[s],
                                     sem_g[s]))
            for s in range(NBUF):
                l = g * NBUF + s
                descs[s].wait()
                gbuf, tbuf = gbufs[s], tbufs[s]

                @plsc.parallel_loop(0, D_MODEL, unroll=8)
                def drow(d, gbuf=gbuf, tbuf=tbuf, l=l):
                    dvec = jnp.full((LANES,), d, jnp.int32)
                    ps = plsc.load_gather(pos_v, [dvec + l * D_MODEL])
                    for kk in range(kb):
                        v = plsc.load_gather(gbuf, [lanes_k[kk], dvec])
                        tbuf[d, pl.ds(kk * LANES, LANES)] = v + ps
                pltpu.async_copy(tbuf, out_hbm.at[l, :, pl.ds(b0, bpw)],
                                 sem_s[s])
            return carry

        lax.fori_loop(0, groups, group, 0)
        for s in range(NBUF):
            pltpu.make_async_copy(
                tbufs[s], out_hbm.at[0, :, pl.ds(0, bpw)], sem_s[s]).wait()

    flat_ids = inputs.reshape(nseq * slen)
    flat_pos = pos_table.reshape(pos_table.shape[0] * D_MODEL)
    raw = k(flat_ids, token_table, flat_pos)
    return jnp.transpose(raw, (2, 0, 1))


# speedup vs baseline: 2.1914x; 2.1914x over previous
import functools

import jax
import jax.numpy as jnp
from jax import lax
from jax.experimental import pallas as pl
from jax.experimental.pallas import tpu as pltpu
from jax.experimental.pallas import tpu_sc as plsc

D_MODEL = 64
NC, NS = 2, 16
NW = NC * NS
NBUF = 4
LANES = 16


def kernel(inputs, token_table, pos_table):
    nseq, slen = inputs.shape
    bpw = nseq // NW
    ids_per_w = bpw * slen
    groups = slen // NBUF
    kb = bpw // LANES
    mesh = plsc.VectorSubcoreMesh(core_axis_name="c", subcore_axis_name="s")

    @functools.partial(
        pl.kernel,
        out_type=jax.ShapeDtypeStruct((slen, D_MODEL, nseq), jnp.float32),
        mesh=mesh,
        scratch_types=(
            [pltpu.VMEM((ids_per_w,), jnp.int32),
             pltpu.VMEM((slen, bpw), jnp.int32),
             pltpu.VMEM((slen * D_MODEL,), jnp.float32)]
            + [pltpu.VMEM((bpw, D_MODEL), jnp.float32)] * NBUF
            + [pltpu.VMEM((D_MODEL, bpw), jnp.float32)] * NBUF
            + [pltpu.SemaphoreType.DMA] * (2 * NBUF)
        ),
        compiler_params=pltpu.CompilerParams(use_tc_tiling_on_sc=False,
                                             needs_layout_passes=False),
    )
    def k(ids_hbm, tok_tab_hbm, pos_tab_hbm, out_hbm, ti, tix, pos_v, *rest):
        gbufs = rest[:NBUF]
        tbufs = rest[NBUF:2 * NBUF]
        sem_g = rest[2 * NBUF:3 * NBUF]
        sem_s = rest[3 * NBUF:]
        wid = lax.axis_index("s") * NC + lax.axis_index("c")
        b0 = wid * bpw

        pltpu.sync_copy(ids_hbm.at[pl.ds(b0 * slen, ids_per_w)], ti)
        pltpu.sync_copy(pos_tab_hbm.at[pl.ds(0, slen * D_MODEL)], pos_v)

        iota = lax.iota(jnp.int32, LANES)
        lanes_k = [iota + LANES * kk for kk in range(kb)]
        seqstr_k = [v * slen for v in lanes_k]
        flatg_k = [v * D_MODEL for v in lanes_k]

        @plsc.parallel_loop(0, slen, unroll=4)
        def idtr(l):
            for kk in range(kb):
                col = plsc.load_gather(ti, [seqstr_k[kk] + l])
                tix[l, pl.ds(kk * LANES, LANES)] = col

        def group(g, carry):
            descs = []
            for s in range(NBUF):
                l = g * NBUF + s

                @pl.when(g > 0)
                def _wait_prev_store(s=s):
                    pltpu.make_async_copy(
                        tbufs[s], out_hbm.at[0, :, pl.ds(0, bpw)],
                        sem_s[s]).wait()

                descs.append(
                    pltpu.async_copy(tok_tab_hbm.at[tix.at[l]], gbufs[s],
                                     sem_g[s]))
            for s in range(NBUF):
                l = g * NBUF + s
                descs[s].wait()
                gbuf, tbuf = gbufs[s], tbufs[s]

                @plsc.parallel_loop(0, D_MODEL, unroll=8)
                def drow(d, gbuf=gbuf, tbuf=tbuf, l=l):
                    rowsel = (jnp.full((LANES,), d, jnp.int32) + iota) & (
                        D_MODEL - 1)
                    ps = plsc.load_gather(pos_v, [rowsel + l * D_MODEL])
                    for kk in range(kb):
                        v = plsc.load_gather(gbuf, [lanes_k[kk], rowsel])
                        plsc.store_scatter(tbuf, [rowsel, lanes_k[kk]],
                                           v + ps)
                pltpu.async_copy(tbuf, out_hbm.at[l, :, pl.ds(b0, bpw)],
                                 sem_s[s])
            return carry

        lax.fori_loop(0, groups, group, 0)
        for s in range(NBUF):
            pltpu.make_async_copy(
                tbufs[s], out_hbm.at[0, :, pl.ds(0, bpw)], sem_s[s]).wait()

    flat_ids = inputs.reshape(nseq * slen)
    flat_pos = pos_table.reshape(pos_table.shape[0] * D_MODEL)
    raw = k(flat_ids, token_table, flat_pos)
    return jnp.transpose(raw, (2, 0, 1))
